# baseline (speedup 1.0000x reference)
import functools

import jax
import jax.numpy as jnp
from jax import lax
from jax.experimental import pallas as pl
from jax.experimental.pallas import tpu as pltpu
from jax.experimental.pallas import tpu_sc as plsc

N_NODES = 10000
N_EDGES = 320000
D_NODE = 128
D_EDGE = 16
D_HID = 64
D_OUT = 16
_HALF = N_EDGES // 2

_NODE_BLK = 1000
_NPB = N_NODES // _NODE_BLK


def _proj_body(nf_ref, w_ref, out_ref):
    out_ref[...] = jnp.dot(nf_ref[...], w_ref[0],
                           preferred_element_type=jnp.float32)


_proj_call = pl.pallas_call(
    _proj_body,
    grid=(2 * _NPB,),
    in_specs=[
        pl.BlockSpec((_NODE_BLK, D_NODE), lambda i: (i % _NPB, 0)),
        pl.BlockSpec((1, D_NODE, D_HID), lambda i: (i // _NPB, 0, 0)),
    ],
    out_specs=pl.BlockSpec((_NODE_BLK, D_HID), lambda i: (i, 0)),
    out_shape=jax.ShapeDtypeStruct((2 * N_NODES, D_HID), jnp.float32),
)

_NC = 2
_NS = 16
_NW = _NC * _NS
_PPW = _HALF // _NW
_PCHUNK = 40
_GROWS = 2 * _PCHUNK
_ITERS = _PPW // _PCHUNK
_IPW = _PPW * 2
_L = 16


@functools.cache
def _make_sc_gather():
    mesh = plsc.VectorSubcoreMesh(core_axis_name="c", subcore_axis_name="s")

    gbuf_t = pltpu.VMEM((_GROWS, D_HID), jnp.float32)
    packed_t = pltpu.VMEM((_PCHUNK, 2 * D_HID), jnp.float32)

    @functools.partial(
        pl.kernel,
        mesh=mesh,
        compiler_params=pltpu.CompilerParams(use_tc_tiling_on_sc=False),
        out_type=jax.ShapeDtypeStruct((_HALF, 2 * D_HID), jnp.float32),
        scratch_types=[
            pltpu.VMEM((_IPW,), jnp.int32),
            pltpu.VMEM((_IPW,), jnp.int32),
            gbuf_t, gbuf_t,
            gbuf_t, gbuf_t,
            packed_t, packed_t,
            pltpu.SemaphoreType.DMA,
            pltpu.SemaphoreType.DMA,
            pltpu.SemaphoreType.DMA,
            pltpu.SemaphoreType.DMA,
        ],
    )
    def _sc_gather(cidx_lo_hbm, cidx_hi_hbm, tab_hbm, pres_hbm,
                   idx_lo, idx_hi, a_lo, a_hi, b_lo, b_hi, pk_a, pk_b,
                   sem_a, sem_b, st_a, st_b):
        wid = lax.axis_index("s") * _NC + lax.axis_index("c")
        base = wid * _PPW
        pltpu.sync_copy(cidx_lo_hbm.at[pl.ds(wid * _IPW, _IPW)], idx_lo)
        pltpu.sync_copy(cidx_hi_hbm.at[pl.ds(wid * _IPW, _IPW)], idx_hi)

        def fire(bufs, sem, i):
            sl = pl.ds(i * _GROWS, _GROWS)
            pltpu.async_copy(tab_hbm.at[idx_lo.at[sl]], bufs[0], sem)
            pltpu.async_copy(tab_hbm.at[idx_hi.at[sl]], bufs[1], sem)

        def wait_gathers(bufs, sem):
            sl = pl.ds(0, _GROWS)
            pltpu.make_async_copy(tab_hbm.at[idx_lo.at[sl]], bufs[0], sem).wait()
            pltpu.make_async_copy(tab_hbm.at[idx_hi.at[sl]], bufs[1], sem).wait()

        def add_pack(bufs, pk):
            d_lo, d_hi = bufs
            for p in range(_PCHUNK):
                for c in range(D_HID // _L):
                    ls = pl.ds(c * _L, _L)
                    pk[p, pl.ds(c * _L, _L)] = (
                        d_lo[p, ls] + d_lo[_PCHUNK + p, ls])
                    pk[p, pl.ds(D_HID + c * _L, _L)] = (
                        d_hi[p, ls] + d_hi[_PCHUNK + p, ls])

        def drain_store(pk, st):
            pltpu.make_async_copy(
                pk, pres_hbm.at[pl.ds(base, _PCHUNK)], st).wait()

        def store(pk, st, i):
            pltpu.async_copy(
                pk, pres_hbm.at[pl.ds(base + i * _PCHUNK, _PCHUNK)], st)

        set_a = (a_lo, a_hi)
        set_b = (b_lo, b_hi)
        fire(set_a, sem_a, 0)

        def body(j, carry):
            fire(set_b, sem_b, 2 * j + 1)
            @pl.when(j > 0)
            def _():
                drain_store(pk_a, st_a)
            wait_gathers(set_a, sem_a)
            add_pack(set_a, pk_a)
            store(pk_a, st_a, 2 * j)
            fire(set_a, sem_a, 2 * j + 2)
            @pl.when(j > 0)
            def _():
                drain_store(pk_b, st_b)
            wait_gathers(set_b, sem_b)
            add_pack(set_b, pk_b)
            store(pk_b, st_b, 2 * j + 1)
            return carry

        lax.fori_loop(0, (_ITERS - 1) // 2, body, 0)

        drain_store(pk_a, st_a)
        wait_gathers(set_a, sem_a)
        add_pack(set_a, pk_a)
        store(pk_a, st_a, _ITERS - 1)
        drain_store(pk_a, st_a)
        drain_store(pk_b, st_b)

    return _sc_gather


_PAIR_BLK = 6400
_N_BLKS = _HALF // _PAIR_BLK


def _mlp_body(eflo_ref, efhi_ref, pres_ref, w1e_ref, b1_ref, w2_ref, b2_ref,
              olo_ref, ohi_ref):
    pres = pres_ref[...]
    w1e = w1e_ref[...]
    b1 = b1_ref[...]
    w2 = w2_ref[...]
    b2 = b2_ref[...]
    dn_in = (((0,), (0,)), ((), ()))
    dn_out = (((0,), (1,)), ((), ()))
    clo = lax.dot_general(eflo_ref[...], w1e, dn_in,
                          preferred_element_type=jnp.float32)
    chi = lax.dot_general(efhi_ref[...], w1e, dn_in,
                          preferred_element_type=jnp.float32)
    hlo = jnp.maximum(clo + pres[:, :D_HID] + b1, 0.0)
    hhi = jnp.maximum(chi + pres[:, D_HID:] + b1, 0.0)
    olo_ref[...] = lax.dot_general(w2, hlo, dn_out,
                                   preferred_element_type=jnp.float32) + b2
    ohi_ref[...] = lax.dot_general(w2, hhi, dn_out,
                                   preferred_element_type=jnp.float32) + b2


_mlp_call = pl.pallas_call(
    _mlp_body,
    grid=(_N_BLKS,),
    in_specs=[
        pl.BlockSpec((D_EDGE, _PAIR_BLK), lambda i: (0, i)),
        pl.BlockSpec((D_EDGE, _PAIR_BLK), lambda i: (0, i + _N_BLKS)),
        pl.BlockSpec((_PAIR_BLK, 2 * D_HID), lambda i: (i, 0)),
        pl.BlockSpec((D_EDGE, D_HID), lambda i: (0, 0)),
        pl.BlockSpec((1, D_HID), lambda i: (0, 0)),
        pl.BlockSpec((D_HID, D_OUT), lambda i: (0, 0)),
        pl.BlockSpec((D_OUT, 1), lambda i: (0, 0)),
    ],
    out_specs=[
        pl.BlockSpec((D_OUT, _PAIR_BLK), lambda i: (0, i)),
        pl.BlockSpec((D_OUT, _PAIR_BLK), lambda i: (0, i)),
    ],
    out_shape=[
        jax.ShapeDtypeStruct((D_OUT, _HALF), jnp.float32),
        jax.ShapeDtypeStruct((D_OUT, _HALF), jnp.float32),
    ],
)


def kernel(edge_index, node_features, edge_features, W1, b1, W2, b2):
    src = edge_index[0].astype(jnp.int32)
    tgt = edge_index[1].astype(jnp.int32) + N_NODES
    w1e = W1[:D_EDGE]
    w_nodes = jnp.stack([W1[D_EDGE:D_EDGE + D_NODE],
                         W1[D_EDGE + D_NODE:]])
    table = _proj_call(node_features, w_nodes)

    def _combined(s, t):
        sr = s.reshape(_NW, _ITERS, _PCHUNK)
        tr = t.reshape(_NW, _ITERS, _PCHUNK)
        return jnp.concatenate(
            [sr[:, :, None, :], tr[:, :, None, :]], axis=2).reshape(-1)

    cidx_lo = _combined(src[:_HALF], tgt[:_HALF])
    cidx_hi = _combined(src[_HALF:], tgt[_HALF:])

    presum2 = _make_sc_gather()(cidx_lo, cidx_hi, table)
    eft = jnp.transpose(edge_features)
    out_lo, out_hi = _mlp_call(
        eft, eft, presum2, w1e, b1.reshape(1, D_HID), W2,
        b2.reshape(D_OUT, 1))
    outt = jnp.concatenate([out_lo, out_hi], axis=1)
    return jnp.transpose(outt)


# speedup vs baseline: 1.4230x; 1.4230x over previous
import functools

import jax
import jax.numpy as jnp
from jax import lax
from jax.experimental import pallas as pl
from jax.experimental.pallas import tpu as pltpu
from jax.experimental.pallas import tpu_sc as plsc

N_NODES = 10000
N_EDGES = 320000
D_NODE = 128
D_EDGE = 16
D_HID = 64
D_OUT = 16
_HALF = N_EDGES // 2

_NODE_BLK = 1000


def _proj_body(nf_ref, ws_ref, wt_ref, ts_ref, tt_ref):
    nf = nf_ref[...]
    ts_ref[...] = jnp.dot(
        nf, ws_ref[...], preferred_element_type=jnp.float32
    ).astype(jnp.bfloat16)
    tt_ref[...] = jnp.dot(
        nf, wt_ref[...], preferred_element_type=jnp.float32
    ).astype(jnp.bfloat16)


_proj_call = pl.pallas_call(
    _proj_body,
    grid=(N_NODES // _NODE_BLK,),
    in_specs=[
        pl.BlockSpec((_NODE_BLK, D_NODE), lambda i: (i, 0)),
        pl.BlockSpec((D_NODE, D_HID), lambda i: (0, 0)),
        pl.BlockSpec((D_NODE, D_HID), lambda i: (0, 0)),
    ],
    out_specs=[
        pl.BlockSpec((_NODE_BLK, D_HID), lambda i: (i, 0)),
        pl.BlockSpec((_NODE_BLK, D_HID), lambda i: (i, 0)),
    ],
    out_shape=[
        jax.ShapeDtypeStruct((N_NODES, D_HID), jnp.bfloat16),
        jax.ShapeDtypeStruct((N_NODES, D_HID), jnp.bfloat16),
    ],
)

_NC = 2
_NS = 16
_NW = _NC * _NS
_PPW = _HALF // _NW
_PCHUNK = 40
_ITERS = _PPW // _PCHUNK
_LB = 32


@functools.cache
def _make_sc_gather():
    mesh = plsc.VectorSubcoreMesh(core_axis_name="c", subcore_axis_name="s")

    row_t = pltpu.VMEM((_PCHUNK, D_HID), jnp.bfloat16)
    packed_t = pltpu.VMEM((_PCHUNK, 2 * D_HID), jnp.float32)

    @functools.partial(
        pl.kernel,
        mesh=mesh,
        compiler_params=pltpu.CompilerParams(
            use_tc_tiling_on_sc=False, needs_layout_passes=False),
        out_type=jax.ShapeDtypeStruct((_HALF, 2 * D_HID), jnp.float32),
        scratch_types=[
            pltpu.VMEM((_PPW,), jnp.int32),
            pltpu.VMEM((_PPW,), jnp.int32),
            pltpu.VMEM((_PPW,), jnp.int32),
            pltpu.VMEM((_PPW,), jnp.int32),
            row_t, row_t, row_t, row_t,
            row_t, row_t, row_t, row_t,
            packed_t, packed_t,
            pltpu.SemaphoreType.DMA,
            pltpu.SemaphoreType.DMA,
            pltpu.SemaphoreType.DMA,
            pltpu.SemaphoreType.DMA,
        ],
    )
    def _sc_gather(src_hbm, tgt_hbm, ts_hbm, tt_hbm, pres_hbm,
                   idx_s_lo, idx_t_lo, idx_s_hi, idx_t_hi,
                   a0, a1, a2, a3, b0, b1_, b2_, b3, pk_a, pk_b,
                   sem_a, sem_b, st_a, st_b):
        wid = lax.axis_index("s") * _NC + lax.axis_index("c")
        base = wid * _PPW
        pltpu.sync_copy(src_hbm.at[pl.ds(base, _PPW)], idx_s_lo)
        pltpu.sync_copy(tgt_hbm.at[pl.ds(base, _PPW)], idx_t_lo)
        pltpu.sync_copy(src_hbm.at[pl.ds(_HALF + base, _PPW)], idx_s_hi)
        pltpu.sync_copy(tgt_hbm.at[pl.ds(_HALF + base, _PPW)], idx_t_hi)

        set_a = (a0, a1, a2, a3)
        set_b = (b0, b1_, b2_, b3)

        def fire(bufs, sem, i):
            sl = pl.ds(i * _PCHUNK, _PCHUNK)
            pltpu.async_copy(ts_hbm.at[idx_s_lo.at[sl]], bufs[0], sem)
            pltpu.async_copy(tt_hbm.at[idx_t_lo.at[sl]], bufs[1], sem)
            pltpu.async_copy(ts_hbm.at[idx_s_hi.at[sl]], bufs[2], sem)
            pltpu.async_copy(tt_hbm.at[idx_t_hi.at[sl]], bufs[3], sem)

        def wait_gathers(bufs, sem):
            sl = pl.ds(0, _PCHUNK)
            pltpu.make_async_copy(ts_hbm.at[idx_s_lo.at[sl]], bufs[0], sem).wait()
            pltpu.make_async_copy(tt_hbm.at[idx_t_lo.at[sl]], bufs[1], sem).wait()
            pltpu.make_async_copy(ts_hbm.at[idx_s_hi.at[sl]], bufs[2], sem).wait()
            pltpu.make_async_copy(tt_hbm.at[idx_t_hi.at[sl]], bufs[3], sem).wait()

        def add_pack(bufs, pk):
            rs_lo, rt_lo, rs_hi, rt_hi = bufs
            for p in range(_PCHUNK):
                for c in range(D_HID // _LB):
                    ls = pl.ds(c * _LB, _LB)
                    pk[p, pl.ds(c * 16, 16)] = plsc.bitcast(
                        rs_lo[p, ls] + rt_lo[p, ls], jnp.float32)
                    pk[p, pl.ds(32 + c * 16, 16)] = plsc.bitcast(
                        rs_hi[p, ls] + rt_hi[p, ls], jnp.float32)

        def drain_store(pk, st):
            pltpu.make_async_copy(
                pk, pres_hbm.at[pl.ds(base, _PCHUNK)], st).wait()

        def store(pk, st, i):
            pltpu.async_copy(
                pk, pres_hbm.at[pl.ds(base + i * _PCHUNK, _PCHUNK)], st)

        fire(set_a, sem_a, 0)

        def body(j, carry):
            fire(set_b, sem_b, 2 * j + 1)
            @pl.when(j > 0)
            def _():
                drain_store(pk_a, st_a)
            wait_gathers(set_a, sem_a)
            add_pack(set_a, pk_a)
            store(pk_a, st_a, 2 * j)
            fire(set_a, sem_a, 2 * j + 2)
            @pl.when(j > 0)
            def _():
                drain_store(pk_b, st_b)
            wait_gathers(set_b, sem_b)
            add_pack(set_b, pk_b)
            store(pk_b, st_b, 2 * j + 1)
            return carry

        lax.fori_loop(0, (_ITERS - 1) // 2, body, 0)

        drain_store(pk_a, st_a)
        wait_gathers(set_a, sem_a)
        add_pack(set_a, pk_a)
        store(pk_a, st_a, _ITERS - 1)
        drain_store(pk_a, st_a)
        drain_store(pk_b, st_b)

    return _sc_gather


_PAIR_BLK = 6400
_N_BLKS = _HALF // _PAIR_BLK


def _mlp_body(eflo_ref, efhi_ref, pres_ref, w1e_a_ref, w1e_b_ref,
              b1_a_ref, b1_b_ref, w2_a_ref, w2_b_ref, b2_ref,
              olo_ref, ohi_ref):
    p4 = pres_ref[...]
    wi = lax.bitcast_convert_type(p4, jnp.int32)
    xa = lax.bitcast_convert_type(wi << 16, jnp.float32)
    xb = lax.bitcast_convert_type(wi & jnp.int32(-65536), jnp.float32)
    pa_lo = xa[:, :32]
    pb_lo = xb[:, :32]
    pa_hi = xa[:, 32:64]
    pb_hi = xb[:, 32:64]
    b1_a = b1_a_ref[...]
    b1_b = b1_b_ref[...]
    b2 = b2_ref[...]
    dn_in = (((0,), (0,)), ((), ()))
    dn_out = (((0,), (1,)), ((), ()))

    def half(ef, pa, pb):
        ca = lax.dot_general(ef, w1e_a_ref[...], dn_in,
                             preferred_element_type=jnp.float32)
        cb = lax.dot_general(ef, w1e_b_ref[...], dn_in,
                             preferred_element_type=jnp.float32)
        ha = jnp.maximum(ca + pa + b1_a, 0.0)
        hb = jnp.maximum(cb + pb + b1_b, 0.0)
        return (lax.dot_general(w2_a_ref[...], ha, dn_out,
                                preferred_element_type=jnp.float32)
                + lax.dot_general(w2_b_ref[...], hb, dn_out,
                                  preferred_element_type=jnp.float32) + b2)

    olo_ref[...] = half(eflo_ref[...], pa_lo, pb_lo)
    ohi_ref[...] = half(efhi_ref[...], pa_hi, pb_hi)


_mlp_call = pl.pallas_call(
    _mlp_body,
    grid=(_N_BLKS,),
    in_specs=[
        pl.BlockSpec((D_EDGE, _PAIR_BLK), lambda i: (0, i)),
        pl.BlockSpec((D_EDGE, _PAIR_BLK), lambda i: (0, i + _N_BLKS)),
        pl.BlockSpec((_PAIR_BLK, 2 * D_HID), lambda i: (i, 0)),
        pl.BlockSpec((D_EDGE, D_HID // 2), lambda i: (0, 0)),
        pl.BlockSpec((D_EDGE, D_HID // 2), lambda i: (0, 0)),
        pl.BlockSpec((1, D_HID // 2), lambda i: (0, 0)),
        pl.BlockSpec((1, D_HID // 2), lambda i: (0, 0)),
        pl.BlockSpec((D_HID // 2, D_OUT), lambda i: (0, 0)),
        pl.BlockSpec((D_HID // 2, D_OUT), lambda i: (0, 0)),
        pl.BlockSpec((D_OUT, 1), lambda i: (0, 0)),
    ],
    out_specs=[
        pl.BlockSpec((D_OUT, _PAIR_BLK), lambda i: (0, i)),
        pl.BlockSpec((D_OUT, _PAIR_BLK), lambda i: (0, i)),
    ],
    out_shape=[
        jax.ShapeDtypeStruct((D_OUT, _HALF), jnp.float32),
        jax.ShapeDtypeStruct((D_OUT, _HALF), jnp.float32),
    ],
)


def kernel(edge_index, node_features, edge_features, W1, b1, W2, b2):
    src = edge_index[0].astype(jnp.int32)
    tgt = edge_index[1].astype(jnp.int32)
    w1e = W1[:D_EDGE]
    w1s = W1[D_EDGE:D_EDGE + D_NODE]
    w1t = W1[D_EDGE + D_NODE:]
    ts, tt = _proj_call(node_features, w1s, w1t)
    presum2 = _make_sc_gather()(src, tgt, ts, tt)
    eft = jnp.transpose(edge_features)
    w1e_a = w1e[:, 0::2]
    w1e_b = w1e[:, 1::2]
    b1_a = b1[0::2].reshape(1, D_HID // 2)
    b1_b = b1[1::2].reshape(1, D_HID // 2)
    w2_a = W2[0::2, :]
    w2_b = W2[1::2, :]
    out_lo, out_hi = _mlp_call(
        eft, eft, presum2, w1e_a, w1e_b, b1_a, b1_b, w2_a, w2_b,
        b2.reshape(D_OUT, 1))
    outt = jnp.concatenate([out_lo, out_hi], axis=1)
    return jnp.transpose(outt)


# speedup vs baseline: 1.5446x; 1.0855x over previous
import functools

import jax
import jax.numpy as jnp
from jax import lax
from jax.experimental import pallas as pl
from jax.experimental.pallas import tpu as pltpu
from jax.experimental.pallas import tpu_sc as plsc

N_NODES = 10000
N_EDGES = 320000
D_NODE = 128
D_EDGE = 16
D_HID = 64
D_OUT = 16
_HALF = N_EDGES // 2

_NODE_BLK = 1000


def _proj_body(nf_ref, ws_ref, wt_ref, ts_ref, tt_ref):
    nf = nf_ref[...]
    ts_ref[...] = jnp.dot(
        nf, ws_ref[...], preferred_element_type=jnp.float32
    ).astype(jnp.bfloat16)
    tt_ref[...] = jnp.dot(
        nf, wt_ref[...], preferred_element_type=jnp.float32
    ).astype(jnp.bfloat16)


_proj_call = pl.pallas_call(
    _proj_body,
    grid=(N_NODES // _NODE_BLK,),
    in_specs=[
        pl.BlockSpec((_NODE_BLK, D_NODE), lambda i: (i, 0)),
        pl.BlockSpec((D_NODE, D_HID), lambda i: (0, 0)),
        pl.BlockSpec((D_NODE, D_HID), lambda i: (0, 0)),
    ],
    out_specs=[
        pl.BlockSpec((_NODE_BLK, D_HID), lambda i: (i, 0)),
        pl.BlockSpec((_NODE_BLK, D_HID), lambda i: (i, 0)),
    ],
    out_shape=[
        jax.ShapeDtypeStruct((N_NODES, D_HID), jnp.bfloat16),
        jax.ShapeDtypeStruct((N_NODES, D_HID), jnp.bfloat16),
    ],
)

_NC = 2
_NS = 16
_NW = _NC * _NS
_PPW = _HALF // _NW
_PCHUNK = 40
_ITERS = _PPW // _PCHUNK
_LB = 32


@functools.cache
def _make_sc_gather(n_pairs):
    ppw = n_pairs // _NW
    iters = ppw // _PCHUNK
    mesh = plsc.VectorSubcoreMesh(core_axis_name="c", subcore_axis_name="s")

    row_t = pltpu.VMEM((_PCHUNK, D_HID), jnp.bfloat16)
    packed_t = pltpu.VMEM((_PCHUNK, 2 * D_HID), jnp.float32)

    @functools.partial(
        pl.kernel,
        mesh=mesh,
        compiler_params=pltpu.CompilerParams(
            use_tc_tiling_on_sc=False, needs_layout_passes=False),
        out_type=jax.ShapeDtypeStruct((n_pairs, 2 * D_HID), jnp.float32),
        scratch_types=[
            pltpu.VMEM((ppw,), jnp.int32),
            pltpu.VMEM((ppw,), jnp.int32),
            pltpu.VMEM((ppw,), jnp.int32),
            pltpu.VMEM((ppw,), jnp.int32),
            row_t, row_t, row_t, row_t,
            row_t, row_t, row_t, row_t,
            packed_t, packed_t,
            pltpu.SemaphoreType.DMA,
            pltpu.SemaphoreType.DMA,
            pltpu.SemaphoreType.DMA,
            pltpu.SemaphoreType.DMA,
        ],
    )
    def _sc_gather(src_lo_hbm, tgt_lo_hbm, src_hi_hbm, tgt_hi_hbm,
                   ts_hbm, tt_hbm, pres_hbm,
                   idx_s_lo, idx_t_lo, idx_s_hi, idx_t_hi,
                   a0, a1, a2, a3, b0, b1_, b2_, b3, pk_a, pk_b,
                   sem_a, sem_b, st_a, st_b):
        wid = lax.axis_index("s") * _NC + lax.axis_index("c")
        base = wid * ppw
        pltpu.sync_copy(src_lo_hbm.at[pl.ds(base, ppw)], idx_s_lo)
        pltpu.sync_copy(tgt_lo_hbm.at[pl.ds(base, ppw)], idx_t_lo)
        pltpu.sync_copy(src_hi_hbm.at[pl.ds(base, ppw)], idx_s_hi)
        pltpu.sync_copy(tgt_hi_hbm.at[pl.ds(base, ppw)], idx_t_hi)

        set_a = (a0, a1, a2, a3)
        set_b = (b0, b1_, b2_, b3)

        def fire(bufs, sem, i):
            sl = pl.ds(i * _PCHUNK, _PCHUNK)
            pltpu.async_copy(ts_hbm.at[idx_s_lo.at[sl]], bufs[0], sem)
            pltpu.async_copy(tt_hbm.at[idx_t_lo.at[sl]], bufs[1], sem)
            pltpu.async_copy(ts_hbm.at[idx_s_hi.at[sl]], bufs[2], sem)
            pltpu.async_copy(tt_hbm.at[idx_t_hi.at[sl]], bufs[3], sem)

        def wait_gathers(bufs, sem):
            sl = pl.ds(0, _PCHUNK)
            pltpu.make_async_copy(ts_hbm.at[idx_s_lo.at[sl]], bufs[0], sem).wait()
            pltpu.make_async_copy(tt_hbm.at[idx_t_lo.at[sl]], bufs[1], sem).wait()
            pltpu.make_async_copy(ts_hbm.at[idx_s_hi.at[sl]], bufs[2], sem).wait()
            pltpu.make_async_copy(tt_hbm.at[idx_t_hi.at[sl]], bufs[3], sem).wait()

        def add_pack(bufs, pk):
            rs_lo, rt_lo, rs_hi, rt_hi = bufs
            for p in range(_PCHUNK):
                for c in range(D_HID // _LB):
                    ls = pl.ds(c * _LB, _LB)
                    pk[p, pl.ds(c * 16, 16)] = plsc.bitcast(
                        rs_lo[p, ls] + rt_lo[p, ls], jnp.float32)
                    pk[p, pl.ds(32 + c * 16, 16)] = plsc.bitcast(
                        rs_hi[p, ls] + rt_hi[p, ls], jnp.float32)

        def drain_store(pk, st):
            pltpu.make_async_copy(
                pk, pres_hbm.at[pl.ds(base, _PCHUNK)], st).wait()

        def store(pk, st, i):
            pltpu.async_copy(
                pk, pres_hbm.at[pl.ds(base + i * _PCHUNK, _PCHUNK)], st)

        fire(set_a, sem_a, 0)
        n_dbl = (iters - 1) // 2

        def body(j, carry):
            fire(set_b, sem_b, 2 * j + 1)
            @pl.when(j > 0)
            def _():
                drain_store(pk_a, st_a)
            wait_gathers(set_a, sem_a)
            add_pack(set_a, pk_a)
            store(pk_a, st_a, 2 * j)
            fire(set_a, sem_a, 2 * j + 2)
            @pl.when(j > 0)
            def _():
                drain_store(pk_b, st_b)
            wait_gathers(set_b, sem_b)
            add_pack(set_b, pk_b)
            store(pk_b, st_b, 2 * j + 1)
            return carry

        lax.fori_loop(0, n_dbl, body, 0)

        if iters % 2 == 1:
            drain_store(pk_a, st_a)
            wait_gathers(set_a, sem_a)
            add_pack(set_a, pk_a)
            store(pk_a, st_a, iters - 1)
        else:
            fire(set_b, sem_b, iters - 1)
            drain_store(pk_a, st_a)
            wait_gathers(set_a, sem_a)
            add_pack(set_a, pk_a)
            store(pk_a, st_a, iters - 2)
            drain_store(pk_b, st_b)
            wait_gathers(set_b, sem_b)
            add_pack(set_b, pk_b)
            store(pk_b, st_b, iters - 1)
        drain_store(pk_a, st_a)
        drain_store(pk_b, st_b)

    return _sc_gather


_PAIR_BLK = 6400
_N_BLKS = _HALF // _PAIR_BLK


def _mlp_body(eflo_ref, efhi_ref, pres_ref, w1e_a_ref, w1e_b_ref,
              b1_a_ref, b1_b_ref, w2_a_ref, w2_b_ref, b2_ref,
              olo_ref, ohi_ref):
    p4 = pres_ref[...]
    wi = lax.bitcast_convert_type(p4, jnp.int32)
    xa = lax.bitcast_convert_type(wi << 16, jnp.float32)
    xb = lax.bitcast_convert_type(wi & jnp.int32(-65536), jnp.float32)
    pa_lo = xa[:, :32]
    pb_lo = xb[:, :32]
    pa_hi = xa[:, 32:64]
    pb_hi = xb[:, 32:64]
    b1_a = b1_a_ref[...]
    b1_b = b1_b_ref[...]
    b2 = b2_ref[...]
    dn_in = (((0,), (0,)), ((), ()))
    dn_out = (((0,), (1,)), ((), ()))

    def half(ef, pa, pb):
        ca = lax.dot_general(ef, w1e_a_ref[...], dn_in,
                             preferred_element_type=jnp.float32)
        cb = lax.dot_general(ef, w1e_b_ref[...], dn_in,
                             preferred_element_type=jnp.float32)
        ha = jnp.maximum(ca + pa + b1_a, 0.0)
        hb = jnp.maximum(cb + pb + b1_b, 0.0)
        return (lax.dot_general(w2_a_ref[...], ha, dn_out,
                                preferred_element_type=jnp.float32)
                + lax.dot_general(w2_b_ref[...], hb, dn_out,
                                  preferred_element_type=jnp.float32) + b2)

    olo_ref[...] = half(eflo_ref[...], pa_lo, pb_lo)
    ohi_ref[...] = half(efhi_ref[...], pa_hi, pb_hi)


@functools.cache
def _make_mlp(n_pairs, lo_blk_off, hi_blk_off):
    n_blks = n_pairs // _PAIR_BLK
    return pl.pallas_call(
        _mlp_body,
        grid=(n_blks,),
        in_specs=[
            pl.BlockSpec((D_EDGE, _PAIR_BLK), lambda i: (0, i + lo_blk_off)),
            pl.BlockSpec((D_EDGE, _PAIR_BLK), lambda i: (0, i + hi_blk_off)),
            pl.BlockSpec((_PAIR_BLK, 2 * D_HID), lambda i: (i, 0)),
            pl.BlockSpec((D_EDGE, D_HID // 2), lambda i: (0, 0)),
            pl.BlockSpec((D_EDGE, D_HID // 2), lambda i: (0, 0)),
            pl.BlockSpec((1, D_HID // 2), lambda i: (0, 0)),
            pl.BlockSpec((1, D_HID // 2), lambda i: (0, 0)),
            pl.BlockSpec((D_HID // 2, D_OUT), lambda i: (0, 0)),
            pl.BlockSpec((D_HID // 2, D_OUT), lambda i: (0, 0)),
            pl.BlockSpec((D_OUT, 1), lambda i: (0, 0)),
        ],
        out_specs=[
            pl.BlockSpec((D_OUT, _PAIR_BLK), lambda i: (0, i)),
            pl.BlockSpec((D_OUT, _PAIR_BLK), lambda i: (0, i)),
        ],
        out_shape=[
            jax.ShapeDtypeStruct((D_OUT, n_pairs), jnp.float32),
            jax.ShapeDtypeStruct((D_OUT, n_pairs), jnp.float32),
        ],
    )


def kernel(edge_index, node_features, edge_features, W1, b1, W2, b2):
    src = edge_index[0].astype(jnp.int32)
    tgt = edge_index[1].astype(jnp.int32)
    w1e = W1[:D_EDGE]
    w1s = W1[D_EDGE:D_EDGE + D_NODE]
    w1t = W1[D_EDGE + D_NODE:]
    ts, tt = _proj_call(node_features, w1s, w1t)
    eft = jnp.transpose(edge_features)
    w1e_a = w1e[:, 0::2]
    w1e_b = w1e[:, 1::2]
    b1_a = b1[0::2].reshape(1, D_HID // 2)
    b1_b = b1[1::2].reshape(1, D_HID // 2)
    w2_a = W2[0::2, :]
    w2_b = W2[1::2, :]
    b2c = b2.reshape(D_OUT, 1)

    slab_a = 76800
    out_parts_lo = []
    out_parts_hi = []
    for start, size in ((0, slab_a), (slab_a, _HALF - slab_a)):
        pres = _make_sc_gather(size)(
            src[start:start + size], tgt[start:start + size],
            src[_HALF + start:_HALF + start + size],
            tgt[_HALF + start:_HALF + start + size], ts, tt)
        o_lo, o_hi = _make_mlp(
            size, start // _PAIR_BLK, (_HALF + start) // _PAIR_BLK)(
            eft, eft, pres, w1e_a, w1e_b, b1_a, b1_b, w2_a, w2_b, b2c)
        out_parts_lo.append(o_lo)
        out_parts_hi.append(o_hi)

    outt = jnp.concatenate(out_parts_lo + out_parts_hi, axis=1)
    return jnp.transpose(outt)


# speedup vs baseline: 1.5734x; 1.0187x over previous
import functools

import jax
import jax.numpy as jnp
from jax import lax
from jax.experimental import pallas as pl
from jax.experimental.pallas import tpu as pltpu
from jax.experimental.pallas import tpu_sc as plsc

N_NODES = 10000
N_EDGES = 320000
D_NODE = 128
D_EDGE = 16
D_HID = 64
D_OUT = 16
_HALF = N_EDGES // 2

_NODE_BLK = 1000


def _proj_body(nf_ref, ws_ref, wt_ref, ts_ref, tt_ref):
    nf = nf_ref[...]
    ts_ref[...] = jnp.dot(
        nf, ws_ref[...], preferred_element_type=jnp.float32
    ).astype(jnp.bfloat16)
    tt_ref[...] = jnp.dot(
        nf, wt_ref[...], preferred_element_type=jnp.float32
    ).astype(jnp.bfloat16)


_proj_call = pl.pallas_call(
    _proj_body,
    grid=(N_NODES // _NODE_BLK,),
    in_specs=[
        pl.BlockSpec((_NODE_BLK, D_NODE), lambda i: (i, 0)),
        pl.BlockSpec((D_NODE, D_HID), lambda i: (0, 0)),
        pl.BlockSpec((D_NODE, D_HID), lambda i: (0, 0)),
    ],
    out_specs=[
        pl.BlockSpec((_NODE_BLK, D_HID), lambda i: (i, 0)),
        pl.BlockSpec((_NODE_BLK, D_HID), lambda i: (i, 0)),
    ],
    out_shape=[
        jax.ShapeDtypeStruct((N_NODES, D_HID), jnp.bfloat16),
        jax.ShapeDtypeStruct((N_NODES, D_HID), jnp.bfloat16),
    ],
)

_NC = 2
_NS = 16
_NW = _NC * _NS
_PPW = _HALF // _NW
_PCHUNK = 40
_ITERS = _PPW // _PCHUNK
_LB = 32


@functools.cache
def _make_sc_gather(n_pairs):
    ppw = n_pairs // _NW
    iters = ppw // _PCHUNK
    mesh = plsc.VectorSubcoreMesh(core_axis_name="c", subcore_axis_name="s")

    row_t = pltpu.VMEM((_PCHUNK, D_HID), jnp.bfloat16)
    packed_t = pltpu.VMEM((_PCHUNK, 2 * D_HID), jnp.float32)

    @functools.partial(
        pl.kernel,
        mesh=mesh,
        compiler_params=pltpu.CompilerParams(
            use_tc_tiling_on_sc=False, needs_layout_passes=False),
        out_type=jax.ShapeDtypeStruct((n_pairs, 2 * D_HID), jnp.float32),
        scratch_types=[
            pltpu.VMEM((ppw,), jnp.int32),
            pltpu.VMEM((ppw,), jnp.int32),
            pltpu.VMEM((ppw,), jnp.int32),
            pltpu.VMEM((ppw,), jnp.int32),
            row_t, row_t, row_t, row_t,
            row_t, row_t, row_t, row_t,
            packed_t, packed_t,
            pltpu.SemaphoreType.DMA,
            pltpu.SemaphoreType.DMA,
            pltpu.SemaphoreType.DMA,
            pltpu.SemaphoreType.DMA,
        ],
    )
    def _sc_gather(src_lo_hbm, tgt_lo_hbm, src_hi_hbm, tgt_hi_hbm,
                   ts_hbm, tt_hbm, pres_hbm,
                   idx_s_lo, idx_t_lo, idx_s_hi, idx_t_hi,
                   a0, a1, a2, a3, b0, b1_, b2_, b3, pk_a, pk_b,
                   sem_a, sem_b, st_a, st_b):
        wid = lax.axis_index("s") * _NC + lax.axis_index("c")
        base = wid * ppw
        pltpu.sync_copy(src_lo_hbm.at[pl.ds(base, ppw)], idx_s_lo)
        pltpu.sync_copy(tgt_lo_hbm.at[pl.ds(base, ppw)], idx_t_lo)
        pltpu.sync_copy(src_hi_hbm.at[pl.ds(base, ppw)], idx_s_hi)
        pltpu.sync_copy(tgt_hi_hbm.at[pl.ds(base, ppw)], idx_t_hi)

        set_a = (a0, a1, a2, a3)
        set_b = (b0, b1_, b2_, b3)

        def fire(bufs, sem, i):
            sl = pl.ds(i * _PCHUNK, _PCHUNK)
            pltpu.async_copy(ts_hbm.at[idx_s_lo.at[sl]], bufs[0], sem)
            pltpu.async_copy(tt_hbm.at[idx_t_lo.at[sl]], bufs[1], sem)
            pltpu.async_copy(ts_hbm.at[idx_s_hi.at[sl]], bufs[2], sem)
            pltpu.async_copy(tt_hbm.at[idx_t_hi.at[sl]], bufs[3], sem)

        def wait_gathers(bufs, sem):
            sl = pl.ds(0, _PCHUNK)
            pltpu.make_async_copy(ts_hbm.at[idx_s_lo.at[sl]], bufs[0], sem).wait()
            pltpu.make_async_copy(tt_hbm.at[idx_t_lo.at[sl]], bufs[1], sem).wait()
            pltpu.make_async_copy(ts_hbm.at[idx_s_hi.at[sl]], bufs[2], sem).wait()
            pltpu.make_async_copy(tt_hbm.at[idx_t_hi.at[sl]], bufs[3], sem).wait()

        def add_pack(bufs, pk):
            rs_lo, rt_lo, rs_hi, rt_hi = bufs
            for p in range(_PCHUNK):
                for c in range(D_HID // _LB):
                    ls = pl.ds(c * _LB, _LB)
                    pk[p, pl.ds(c * 16, 16)] = plsc.bitcast(
                        rs_lo[p, ls] + rt_lo[p, ls], jnp.float32)
                    pk[p, pl.ds(32 + c * 16, 16)] = plsc.bitcast(
                        rs_hi[p, ls] + rt_hi[p, ls], jnp.float32)

        def drain_store(pk, st):
            pltpu.make_async_copy(
                pk, pres_hbm.at[pl.ds(base, _PCHUNK)], st).wait()

        def store(pk, st, i):
            pltpu.async_copy(
                pk, pres_hbm.at[pl.ds(base + i * _PCHUNK, _PCHUNK)], st)

        fire(set_a, sem_a, 0)
        n_dbl = (iters - 1) // 2

        def body(j, carry):
            fire(set_b, sem_b, 2 * j + 1)
            @pl.when(j > 0)
            def _():
                drain_store(pk_a, st_a)
            wait_gathers(set_a, sem_a)
            add_pack(set_a, pk_a)
            store(pk_a, st_a, 2 * j)
            fire(set_a, sem_a, 2 * j + 2)
            @pl.when(j > 0)
            def _():
                drain_store(pk_b, st_b)
            wait_gathers(set_b, sem_b)
            add_pack(set_b, pk_b)
            store(pk_b, st_b, 2 * j + 1)
            return carry

        lax.fori_loop(0, n_dbl, body, 0)

        if iters % 2 == 1:
            drain_store(pk_a, st_a)
            wait_gathers(set_a, sem_a)
            add_pack(set_a, pk_a)
            store(pk_a, st_a, iters - 1)
        else:
            fire(set_b, sem_b, iters - 1)
            drain_store(pk_a, st_a)
            wait_gathers(set_a, sem_a)
            add_pack(set_a, pk_a)
            store(pk_a, st_a, iters - 2)
            drain_store(pk_b, st_b)
            wait_gathers(set_b, sem_b)
            add_pack(set_b, pk_b)
            store(pk_b, st_b, iters - 1)
        drain_store(pk_a, st_a)
        drain_store(pk_b, st_b)

    return _sc_gather


_PAIR_BLK = 6400
_N_BLKS = _HALF // _PAIR_BLK


def _mlp_body(eflo_ref, efhi_ref, pres_ref, w1e_a_ref, w1e_b_ref,
              b1_a_ref, b1_b_ref, w2_a_ref, w2_b_ref, b2_ref,
              olo_ref, ohi_ref):
    p4 = pres_ref[...]
    wi = lax.bitcast_convert_type(p4, jnp.int32)
    xa = lax.bitcast_convert_type(wi << 16, jnp.float32)
    xb = lax.bitcast_convert_type(wi & jnp.int32(-65536), jnp.float32)
    pa_lo = xa[:, :32]
    pb_lo = xb[:, :32]
    pa_hi = xa[:, 32:64]
    pb_hi = xb[:, 32:64]
    b1_a = b1_a_ref[...]
    b1_b = b1_b_ref[...]
    b2 = b2_ref[...]
    dn_in = (((0,), (0,)), ((), ()))
    dn_out = (((0,), (1,)), ((), ()))

    def half(ef, pa, pb):
        ca = lax.dot_general(ef, w1e_a_ref[...], dn_in,
                             preferred_element_type=jnp.float32)
        cb = lax.dot_general(ef, w1e_b_ref[...], dn_in,
                             preferred_element_type=jnp.float32)
        ha = jnp.maximum(ca + pa + b1_a, 0.0)
        hb = jnp.maximum(cb + pb + b1_b, 0.0)
        return (lax.dot_general(w2_a_ref[...], ha, dn_out,
                                preferred_element_type=jnp.float32)
                + lax.dot_general(w2_b_ref[...], hb, dn_out,
                                  preferred_element_type=jnp.float32) + b2)

    olo_ref[...] = half(eflo_ref[...], pa_lo, pb_lo)
    ohi_ref[...] = half(efhi_ref[...], pa_hi, pb_hi)


@functools.cache
def _make_mlp(n_pairs, lo_blk_off, hi_blk_off):
    n_blks = n_pairs // _PAIR_BLK
    return pl.pallas_call(
        _mlp_body,
        grid=(n_blks,),
        in_specs=[
            pl.BlockSpec((D_EDGE, _PAIR_BLK), lambda i: (0, i + lo_blk_off)),
            pl.BlockSpec((D_EDGE, _PAIR_BLK), lambda i: (0, i + hi_blk_off)),
            pl.BlockSpec((_PAIR_BLK, 2 * D_HID), lambda i: (i, 0)),
            pl.BlockSpec((D_EDGE, D_HID // 2), lambda i: (0, 0)),
            pl.BlockSpec((D_EDGE, D_HID // 2), lambda i: (0, 0)),
            pl.BlockSpec((1, D_HID // 2), lambda i: (0, 0)),
            pl.BlockSpec((1, D_HID // 2), lambda i: (0, 0)),
            pl.BlockSpec((D_HID // 2, D_OUT), lambda i: (0, 0)),
            pl.BlockSpec((D_HID // 2, D_OUT), lambda i: (0, 0)),
            pl.BlockSpec((D_OUT, 1), lambda i: (0, 0)),
        ],
        out_specs=[
            pl.BlockSpec((D_OUT, _PAIR_BLK), lambda i: (0, i)),
            pl.BlockSpec((D_OUT, _PAIR_BLK), lambda i: (0, i)),
        ],
        out_shape=[
            jax.ShapeDtypeStruct((D_OUT, n_pairs), jnp.float32),
            jax.ShapeDtypeStruct((D_OUT, n_pairs), jnp.float32),
        ],
    )


def kernel(edge_index, node_features, edge_features, W1, b1, W2, b2):
    src = edge_index[0].astype(jnp.int32)
    tgt = edge_index[1].astype(jnp.int32)
    w1e = W1[:D_EDGE]
    w1s = W1[D_EDGE:D_EDGE + D_NODE]
    w1t = W1[D_EDGE + D_NODE:]
    ts, tt = _proj_call(node_features, w1s, w1t)
    eft = jnp.transpose(edge_features)
    w1e_a = w1e[:, 0::2]
    w1e_b = w1e[:, 1::2]
    b1_a = b1[0::2].reshape(1, D_HID // 2)
    b1_b = b1[1::2].reshape(1, D_HID // 2)
    w2_a = W2[0::2, :]
    w2_b = W2[1::2, :]
    b2c = b2.reshape(D_OUT, 1)

    slabs = ((0, 51200), (51200, 51200), (102400, 57600))
    out_parts_lo = []
    out_parts_hi = []
    for start, size in slabs:
        pres = _make_sc_gather(size)(
            src[start:start + size], tgt[start:start + size],
            src[_HALF + start:_HALF + start + size],
            tgt[_HALF + start:_HALF + start + size], ts, tt)
        o_lo, o_hi = _make_mlp(
            size, start // _PAIR_BLK, (_HALF + start) // _PAIR_BLK)(
            eft, eft, pres, w1e_a, w1e_b, b1_a, b1_b, w2_a, w2_b, b2c)
        out_parts_lo.append(o_lo)
        out_parts_hi.append(o_hi)

    outt = jnp.concatenate(out_parts_lo + out_parts_hi, axis=1)
    return jnp.transpose(outt)
